# Initial kernel scaffold; baseline (speedup 1.0000x reference)
import jax
import jax.numpy as jnp
from jax.experimental import pallas as pl


def kernel(node_features, edge_features, edge_index, We1, be1, We2, be2, ge, bge, Wn1, bn1, Wn2, bn2, gn, bgn):
    raise NotImplementedError("write your pallas kernel here")


# speedup vs baseline: 3.2801x; 3.2801x over previous
import functools

import jax
import jax.numpy as jnp
from jax import lax
from jax.experimental import pallas as pl
from jax.experimental.pallas import tpu as pltpu
from jax.experimental.pallas import tpu_sc as plsc

N_NODES = 10000
N_EDGES = 160000
D = 128

NC, NS = 2, 16
NW = NC * NS
CH = 128
NCHUNK = N_EDGES // CH
CPW = -(-NCHUNK // NW)
RPT = 640
RPT_LAST = N_NODES - (NS - 1) * RPT


def _edge_mlp_body(e_ref, ps_ref, qd_ref, w1_ref, w2_ref, pr_ref, o_ref):
    e = e_ref[...]
    h = jnp.dot(e, w1_ref[...], preferred_element_type=jnp.float32)
    h = h + ps_ref[...] + qd_ref[...] + pr_ref[0, :][None, :]
    h = h * jax.nn.sigmoid(h)
    h = jnp.dot(h, w2_ref[...], preferred_element_type=jnp.float32)
    h = h + pr_ref[1, :][None, :]
    mu = jnp.mean(h, axis=-1, keepdims=True)
    var = jnp.mean((h - mu) ** 2, axis=-1, keepdims=True)
    h = (h - mu) * lax.rsqrt(var + 1e-5)
    o_ref[...] = h * pr_ref[2, :][None, :] + pr_ref[3, :][None, :] + e


def _edge_mlp(e, ps, qd, w1a, w2, params, block=2000):
    grid = (N_EDGES // block,)
    row = pl.BlockSpec((block, D), lambda i: (i, 0))
    full = lambda shape: pl.BlockSpec(shape, lambda i: (0,) * len(shape))
    return pl.pallas_call(
        _edge_mlp_body,
        grid=grid,
        in_specs=[row, row, row, full((D, D)), full((D, D)), full((4, D))],
        out_specs=row,
        out_shape=jax.ShapeDtypeStruct((N_EDGES, D), jnp.float32),
    )(e, ps, qd, w1a, w2, params)


def _node_mlp_body(x_ref, a0_ref, a1_ref, w1x_ref, w1a_ref, w2_ref, pr_ref,
                   o_ref):
    x = x_ref[...]
    a = a0_ref[...] + a1_ref[...]
    h = (jnp.dot(x, w1x_ref[...], preferred_element_type=jnp.float32)
         + jnp.dot(a, w1a_ref[...], preferred_element_type=jnp.float32)
         + pr_ref[0, :][None, :])
    h = h * jax.nn.sigmoid(h)
    h = jnp.dot(h, w2_ref[...], preferred_element_type=jnp.float32)
    h = h + pr_ref[1, :][None, :]
    mu = jnp.mean(h, axis=-1, keepdims=True)
    var = jnp.mean((h - mu) ** 2, axis=-1, keepdims=True)
    h = (h - mu) * lax.rsqrt(var + 1e-5)
    o_ref[...] = h * pr_ref[2, :][None, :] + pr_ref[3, :][None, :] + x


def _node_mlp(x, a0, a1, w1x, w1a, w2, params, block=2000):
    grid = (N_NODES // block,)
    row = pl.BlockSpec((block, D), lambda i: (i, 0))
    full = lambda shape: pl.BlockSpec(shape, lambda i: (0,) * len(shape))
    return pl.pallas_call(
        _node_mlp_body,
        grid=grid,
        in_specs=[row, row, row, full((D, D)), full((D, D)), full((D, D)),
                  full((4, D))],
        out_specs=row,
        out_shape=jax.ShapeDtypeStruct((N_NODES, D), jnp.float32),
    )(x, a0, a1, w1x, w1a, w2, params)


def _pq_body(x_ref, w_ref, p_ref, q_ref):
    h = jnp.dot(x_ref[...], w_ref[...], preferred_element_type=jnp.float32)
    p_ref[...] = h[:, :D]
    q_ref[...] = h[:, D:]


def _pq(x, w1bc, block=2000):
    grid = (N_NODES // block,)
    row = pl.BlockSpec((block, D), lambda i: (i, 0))
    return pl.pallas_call(
        _pq_body,
        grid=grid,
        in_specs=[row, pl.BlockSpec((D, 2 * D), lambda i: (0, 0))],
        out_specs=[row, row],
        out_shape=[jax.ShapeDtypeStruct((N_NODES, D), jnp.float32),
                   jax.ShapeDtypeStruct((N_NODES, D), jnp.float32)],
    )(x, w1bc)


def _gather_body(p_hbm, q_hbm, src_hbm, dst_hbm, ops_hbm, oqd_hbm,
                 idx_s, idx_d, bufp, bufq, semp, semq):
    c = lax.axis_index("c")
    s = lax.axis_index("s")
    wid = c * NS + s

    def body(j, carry):
        cid = wid + j * NW

        @pl.when(cid < NCHUNK)
        def _():
            pltpu.sync_copy(src_hbm.at[pl.ds(cid * CH, CH)], idx_s)
            pltpu.sync_copy(dst_hbm.at[pl.ds(cid * CH, CH)], idx_d)
            cp = pltpu.async_copy(p_hbm.at[idx_s], bufp, semp)
            cq = pltpu.async_copy(q_hbm.at[idx_d], bufq, semq)
            cp.wait()
            cq.wait()
            off = cid * CH
            pltpu.sync_copy(bufp, ops_hbm.at[pl.ds(off, CH)])
            pltpu.sync_copy(bufq, oqd_hbm.at[pl.ds(off, CH)])

        return carry

    lax.fori_loop(0, CPW, body, 0)


def _scatter_kernel_body(e_hbm, dst_hbm, zero_hbm, out_hbm, acc_sh, idx_v,
                         rows_v):
    c = lax.axis_index("c")
    s = lax.axis_index("s")
    wid = c * NS + s

    @pl.when(s < NS - 1)
    def _():
        pltpu.sync_copy(zero_hbm.at[pl.ds(s * RPT, RPT)],
                        acc_sh.at[pl.ds(s * RPT, RPT)])

    @pl.when(s == NS - 1)
    def _():
        pltpu.sync_copy(zero_hbm.at[pl.ds((NS - 1) * RPT, RPT_LAST)],
                        acc_sh.at[pl.ds((NS - 1) * RPT, RPT_LAST)])

    plsc.subcore_barrier()

    def body(j, carry):
        cid = wid + j * NW

        @pl.when(cid < NCHUNK)
        def _():
            pltpu.sync_copy(dst_hbm.at[pl.ds(cid * CH, CH)], idx_v)
            pltpu.sync_copy(e_hbm.at[pl.ds(cid * CH, CH)], rows_v)
            pltpu.sync_copy(rows_v, acc_sh.at[idx_v], add=True)

        return carry

    lax.fori_loop(0, CPW, body, 0)
    plsc.subcore_barrier()

    @pl.when(s < NS - 1)
    def _():
        pltpu.sync_copy(acc_sh.at[pl.ds(s * RPT, RPT)],
                        out_hbm.at[c, pl.ds(s * RPT, RPT)])

    @pl.when(s == NS - 1)
    def _():
        pltpu.sync_copy(acc_sh.at[pl.ds((NS - 1) * RPT, RPT_LAST)],
                        out_hbm.at[c, pl.ds((NS - 1) * RPT, RPT_LAST)])


@functools.lru_cache(maxsize=None)
def _sc_kernels():
    mesh = plsc.VectorSubcoreMesh(core_axis_name="c", subcore_axis_name="s")
    gather = pl.kernel(
        _gather_body,
        out_type=(jax.ShapeDtypeStruct((N_EDGES, D), jnp.float32),
                  jax.ShapeDtypeStruct((N_EDGES, D), jnp.float32)),
        mesh=mesh,
        scratch_types=[
            pltpu.VMEM((CH,), jnp.int32),
            pltpu.VMEM((CH,), jnp.int32),
            pltpu.VMEM((CH, D), jnp.float32),
            pltpu.VMEM((CH, D), jnp.float32),
            pltpu.SemaphoreType.DMA,
            pltpu.SemaphoreType.DMA,
        ],
    )
    scatter = pl.kernel(
        _scatter_kernel_body,
        out_type=jax.ShapeDtypeStruct((NC, N_NODES, D), jnp.float32),
        mesh=mesh,
        scratch_types=[
            pltpu.VMEM_SHARED((N_NODES, D), jnp.float32),
            pltpu.VMEM((CH,), jnp.int32),
            pltpu.VMEM((CH, D), jnp.float32),
        ],
    )
    return gather, scatter


def kernel(node_features, edge_features, edge_index, We1, be1, We2, be2, ge,
           bge, Wn1, bn1, Wn2, bn2, gn, bgn):
    x = node_features
    e = edge_features
    src1 = edge_index[0]
    dst1 = edge_index[1]
    zeros_nd = jnp.zeros((N_NODES, D), jnp.float32)
    for i in range(We1.shape[0]):
        w1a = We1[i, :D]
        w1bc = jnp.concatenate([We1[i, D:2 * D], We1[i, 2 * D:]], axis=1)
        sc_gather, sc_scatter = _sc_kernels()
        p, q = _pq(x, w1bc)
        ps, qd = sc_gather(p, q, src1, dst1)
        eparams = jnp.stack([be1[i], be2[i], ge[i], bge[i]])
        e = _edge_mlp(e, ps, qd, w1a, We2[i], eparams)
        agg2 = sc_scatter(e, dst1, zeros_nd)
        nparams = jnp.stack([bn1[i], bn2[i], gn[i], bgn[i]])
        x = _node_mlp(x, agg2[0], agg2[1], Wn1[i, :D], Wn1[i, D:], Wn2[i],
                      nparams)
    return x
